# baseline (speedup 1.0000x reference)
import functools

import jax
import jax.numpy as jnp
from jax import lax
from jax.experimental import pallas as pl
from jax.experimental.pallas import tpu as pltpu
from jax.experimental.pallas import tpu_sc as plsc

IN_F = 16384
OUT_F = 16384
NNZ = 268435
BATCH = 256

N_TILES = 16
CHUNK = 64
N_CHUNKS = -(-NNZ // (N_TILES * CHUNK * 3)) * 3
NNZ_PAD = N_TILES * CHUNK * N_CHUNKS
QCOLS = 64
ROWS_PER_TILE = OUT_F // N_TILES


def _sc_body(x4_hbm, cols_hbm, rows_hbm, vals_hbm, bias_hbm, out_hbm,
             acc, colsv, rowsv, valsv, g0, g1, g2, sg0, sg1, sg2):
    c = lax.axis_index("c")
    s = lax.axis_index("s")

    pltpu.sync_copy(cols_hbm.at[s], colsv)
    pltpu.sync_copy(rows_hbm.at[s], rowsv)
    pltpu.sync_copy(vals_hbm.at[s], valsv)

    rslice = pl.ds(s * ROWS_PER_TILE, ROWS_PER_TILE)

    def scale(i, g):
        for k16 in range(0, CHUNK, 16):
            vvec = valsv[i, pl.ds(k16, 16)]
            for m8 in range(0, 16, 8):
                prods = []
                for t in range(8):
                    v = vvec[m8 + t]
                    for j in range(QCOLS // 16):
                        sl = (k16 + m8 + t, pl.ds(j * 16, 16))
                        prods.append((sl, g[sl] * v))
                for sl, p in prods:
                    g[sl] = p

    for sub in range(2):
        q = 2 * c + sub
        xq = x4_hbm.at[q]

        def gather_start(i, g, sem):
            pltpu.async_copy(xq.at[colsv.at[i]], g, sem)

        def gather_wait(i, g, sem):
            pltpu.make_async_copy(xq.at[colsv.at[i]], g, sem).wait()

        pltpu.sync_copy(bias_hbm.at[rslice], acc.at[rslice])
        plsc.subcore_barrier()

        gather_start(0, g0, sg0)
        gather_start(1, g1, sg1)

        @pl.loop(0, N_CHUNKS, step=3)
        def _(i):
            gather_start(i + 2, g2, sg2)
            gather_wait(i, g0, sg0)
            scale(i, g0)
            pltpu.sync_copy(g0, acc.at[rowsv.at[i]], add=True)
            @pl.when(i + 3 < N_CHUNKS)
            def _():
                gather_start(i + 3, g0, sg0)
            gather_wait(i + 1, g1, sg1)
            scale(i + 1, g1)
            pltpu.sync_copy(g1, acc.at[rowsv.at[i + 1]], add=True)
            @pl.when(i + 4 < N_CHUNKS)
            def _():
                gather_start(i + 4, g1, sg1)
            gather_wait(i + 2, g2, sg2)
            scale(i + 2, g2)
            pltpu.sync_copy(g2, acc.at[rowsv.at[i + 2]], add=True)

        plsc.subcore_barrier()
        pltpu.sync_copy(acc.at[rslice], out_hbm.at[q, rslice])
        plsc.subcore_barrier()


@jax.jit
def kernel(x, values, bias, rows, cols):
    x4 = x.reshape(IN_F, 4, QCOLS).transpose(1, 0, 2)
    pad = NNZ_PAD - NNZ
    cols_t = jnp.pad(cols.astype(jnp.int32), (0, pad)).reshape(
        N_TILES, N_CHUNKS, CHUNK)
    rows_t = jnp.pad(rows.astype(jnp.int32), (0, pad)).reshape(
        N_TILES, N_CHUNKS, CHUNK)
    vals_t = jnp.pad(values, (0, pad)).reshape(N_TILES, N_CHUNKS, CHUNK)
    bias_b = jnp.broadcast_to(bias.astype(jnp.float32), (OUT_F, QCOLS))

    mesh = plsc.VectorSubcoreMesh(core_axis_name="c", subcore_axis_name="s")
    run = pl.kernel(
        _sc_body,
        out_type=jax.ShapeDtypeStruct((4, OUT_F, QCOLS), jnp.float32),
        mesh=mesh,
        compiler_params=pltpu.CompilerParams(use_tc_tiling_on_sc=False),
        scratch_types=[
            pltpu.VMEM_SHARED((OUT_F, QCOLS), jnp.float32),
            pltpu.VMEM((N_CHUNKS, CHUNK), jnp.int32),
            pltpu.VMEM((N_CHUNKS, CHUNK), jnp.int32),
            pltpu.VMEM((N_CHUNKS, CHUNK), jnp.float32),
            pltpu.VMEM((CHUNK, QCOLS), jnp.float32),
            pltpu.VMEM((CHUNK, QCOLS), jnp.float32),
            pltpu.VMEM((CHUNK, QCOLS), jnp.float32),
            pltpu.SemaphoreType.DMA,
            pltpu.SemaphoreType.DMA,
            pltpu.SemaphoreType.DMA,
        ],
    )
    out4 = run(x4, cols_t, rows_t, vals_t, bias_b)
    return out4.transpose(1, 0, 2).reshape(OUT_F, BATCH)


# speedup vs baseline: 1.0183x; 1.0183x over previous
import functools

import jax
import jax.numpy as jnp
from jax import lax
from jax.experimental import pallas as pl
from jax.experimental.pallas import tpu as pltpu
from jax.experimental.pallas import tpu_sc as plsc

IN_F = 16384
OUT_F = 16384
NNZ = 268435
BATCH = 256

N_TILES = 16
CHUNK = 64
N_CHUNKS = -(-NNZ // (N_TILES * CHUNK * 3)) * 3
NNZ_PAD = N_TILES * CHUNK * N_CHUNKS
QCOLS = 64
ROWS_PER_TILE = OUT_F // N_TILES


def _sc_body(x4_hbm, cols_hbm, rows_hbm, vals_hbm, bias_hbm, out_hbm,
             acc, colsv, rowsv, valsv, g0, g1, g2, sg0, sg1, sg2):
    c = lax.axis_index("c")
    s = lax.axis_index("s")

    pltpu.sync_copy(cols_hbm.at[s], colsv)
    pltpu.sync_copy(rows_hbm.at[s], rowsv)
    pltpu.sync_copy(vals_hbm.at[s], valsv)

    rslice = pl.ds(s * ROWS_PER_TILE, ROWS_PER_TILE)

    def scale(i, g):
        @pl.loop(0, CHUNK, step=16)
        def _(k16):
            vvec = valsv[i, pl.ds(k16, 16)]
            for m8 in range(0, 16, 8):
                prods = []
                for t in range(8):
                    v = vvec[m8 + t]
                    for j in range(QCOLS // 16):
                        sl = (k16 + m8 + t, pl.ds(j * 16, 16))
                        prods.append((sl, g[sl] * v))
                for sl, p in prods:
                    g[sl] = p

    for sub in range(2):
        q = 2 * c + sub
        xq = x4_hbm.at[q]

        def gather_start(i, g, sem):
            pltpu.async_copy(xq.at[colsv.at[i]], g, sem)

        def gather_wait(i, g, sem):
            pltpu.make_async_copy(xq.at[colsv.at[i]], g, sem).wait()

        pltpu.sync_copy(bias_hbm.at[rslice], acc.at[rslice])
        plsc.subcore_barrier()

        gather_start(0, g0, sg0)
        gather_start(1, g1, sg1)

        @pl.loop(0, N_CHUNKS, step=3)
        def _(i):
            gather_start(i + 2, g2, sg2)
            gather_wait(i, g0, sg0)
            scale(i, g0)
            pltpu.sync_copy(g0, acc.at[rowsv.at[i]], add=True)
            @pl.when(i + 3 < N_CHUNKS)
            def _():
                gather_start(i + 3, g0, sg0)
            gather_wait(i + 1, g1, sg1)
            scale(i + 1, g1)
            pltpu.sync_copy(g1, acc.at[rowsv.at[i + 1]], add=True)
            @pl.when(i + 4 < N_CHUNKS)
            def _():
                gather_start(i + 4, g1, sg1)
            gather_wait(i + 2, g2, sg2)
            scale(i + 2, g2)
            pltpu.sync_copy(g2, acc.at[rowsv.at[i + 2]], add=True)

        plsc.subcore_barrier()
        pltpu.sync_copy(acc.at[rslice], out_hbm.at[q, rslice])
        plsc.subcore_barrier()


@jax.jit
def kernel(x, values, bias, rows, cols):
    x4 = x.reshape(IN_F, 4, QCOLS).transpose(1, 0, 2)
    pad = NNZ_PAD - NNZ
    cols_t = jnp.pad(cols.astype(jnp.int32), (0, pad)).reshape(
        N_TILES, N_CHUNKS, CHUNK)
    rows_t = jnp.pad(rows.astype(jnp.int32), (0, pad)).reshape(
        N_TILES, N_CHUNKS, CHUNK)
    vals_t = jnp.pad(values, (0, pad)).reshape(N_TILES, N_CHUNKS, CHUNK)
    bias_b = jnp.broadcast_to(bias.astype(jnp.float32), (OUT_F, QCOLS))

    mesh = plsc.VectorSubcoreMesh(core_axis_name="c", subcore_axis_name="s")
    run = pl.kernel(
        _sc_body,
        out_type=jax.ShapeDtypeStruct((4, OUT_F, QCOLS), jnp.float32),
        mesh=mesh,
        compiler_params=pltpu.CompilerParams(use_tc_tiling_on_sc=False),
        scratch_types=[
            pltpu.VMEM_SHARED((OUT_F, QCOLS), jnp.float32),
            pltpu.VMEM((N_CHUNKS, CHUNK), jnp.int32),
            pltpu.VMEM((N_CHUNKS, CHUNK), jnp.int32),
            pltpu.VMEM((N_CHUNKS, CHUNK), jnp.float32),
            pltpu.VMEM((CHUNK, QCOLS), jnp.float32),
            pltpu.VMEM((CHUNK, QCOLS), jnp.float32),
            pltpu.VMEM((CHUNK, QCOLS), jnp.float32),
            pltpu.SemaphoreType.DMA,
            pltpu.SemaphoreType.DMA,
            pltpu.SemaphoreType.DMA,
        ],
    )
    out4 = run(x4, cols_t, rows_t, vals_t, bias_b)
    return out4.transpose(1, 0, 2).reshape(OUT_F, BATCH)
